# Initial kernel scaffold; baseline (speedup 1.0000x reference)
import jax
import jax.numpy as jnp
from jax.experimental import pallas as pl


def kernel(bits, constellation):
    raise NotImplementedError("write your pallas kernel here")


# speedup vs baseline: 53.9251x; 53.9251x over previous
import functools

import jax
import jax.numpy as jnp
from jax import lax
from jax.experimental import pallas as pl
from jax.experimental.pallas import tpu as pltpu
from jax.experimental.pallas import tpu_sc as plsc

_LANES = 16


@functools.lru_cache(maxsize=None)
def _build(batch, nbits):
    info = plsc.get_sparse_core_info()
    num_cores, num_subcores = info.num_cores, info.num_subcores
    nw = num_cores * num_subcores
    syms = nbits // 10
    assert batch % nw == 0
    rows_per_w = batch // nw
    groups = syms // _LANES

    mesh = plsc.VectorSubcoreMesh(core_axis_name="c", subcore_axis_name="s")

    @functools.partial(
        pl.kernel,
        mesh=mesh,
        out_type=jax.ShapeDtypeStruct((batch, 2 * syms), jnp.float32),
        compiler_params=pltpu.CompilerParams(needs_layout_passes=False),
        scratch_types=[
            pltpu.VMEM((nbits,), jnp.int32),
            pltpu.VMEM((2 * syms,), jnp.float32),
            pltpu.VMEM((2048,), jnp.float32),
        ],
    )
    def qam_kernel(bits_hbm, const_hbm, out_hbm, bits_v, out_v, const_v):
        wid = lax.axis_index("s") * num_cores + lax.axis_index("c")
        pltpu.sync_copy(const_hbm, const_v)
        ten_iota = lax.iota(jnp.int32, _LANES) * 10

        def row_body(r, carry):
            row = wid * rows_per_w + r
            pltpu.sync_copy(bits_hbm.at[row], bits_v)

            def grp_body(g, c):
                idx0 = ten_iota + g * (10 * _LANES)
                full = plsc.load_gather(bits_v, [idx0])
                for j in range(1, 10):
                    bj = plsc.load_gather(bits_v, [idx0 + j])
                    full = full * 2 + bj
                re = plsc.load_gather(const_v, [full * 2])
                im = plsc.load_gather(const_v, [full * 2 + 1])
                out_v[pl.ds(g * _LANES, _LANES)] = re
                out_v[pl.ds(syms + g * _LANES, _LANES)] = im
                return c

            lax.fori_loop(0, groups, grp_body, 0)
            pltpu.sync_copy(out_v, out_hbm.at[row])
            return carry

        lax.fori_loop(0, rows_per_w, row_body, 0)

    return qam_kernel


def kernel(bits, constellation):
    bits = bits.astype(jnp.int32)
    batch, nbits = bits.shape
    syms = nbits // 10
    const_flat = constellation.astype(jnp.float32).reshape(-1)
    out = _build(batch, nbits)(bits, const_flat)
    return out.reshape(batch, 2, syms)


# speedup vs baseline: 115.1393x; 2.1352x over previous
import functools

import jax
import jax.numpy as jnp
from jax import lax
from jax.experimental import pallas as pl
from jax.experimental.pallas import tpu as pltpu
from jax.experimental.pallas import tpu_sc as plsc

_LANES = 16


@functools.lru_cache(maxsize=None)
def _build(batch, nbits):
    info = plsc.get_sparse_core_info()
    num_cores, num_subcores = info.num_cores, info.num_subcores
    nw = num_cores * num_subcores
    syms = nbits // 10
    assert batch % (2 * nw) == 0
    rows_per_w = batch // nw
    groups = syms // _LANES

    mesh = plsc.VectorSubcoreMesh(core_axis_name="c", subcore_axis_name="s")

    @functools.partial(
        pl.kernel,
        mesh=mesh,
        out_type=jax.ShapeDtypeStruct((batch, 2 * syms), jnp.float32),
        compiler_params=pltpu.CompilerParams(needs_layout_passes=False),
        scratch_types=[
            pltpu.VMEM((nbits,), jnp.int32),
            pltpu.VMEM((nbits,), jnp.int32),
            pltpu.VMEM((2 * syms,), jnp.float32),
            pltpu.VMEM((2 * syms,), jnp.float32),
            pltpu.VMEM((2048,), jnp.float32),
            pltpu.SemaphoreType.DMA,
            pltpu.SemaphoreType.DMA,
            pltpu.SemaphoreType.DMA,
            pltpu.SemaphoreType.DMA,
        ],
    )
    def qam_kernel(bits_hbm, const_hbm, out_hbm, bits_v0, bits_v1, out_v0,
                   out_v1, const_v, in_sem0, in_sem1, out_sem0, out_sem1):
        wid = lax.axis_index("s") * num_cores + lax.axis_index("c")
        row0 = wid * rows_per_w
        pltpu.sync_copy(const_hbm, const_v)
        ten_iota = lax.iota(jnp.int32, _LANES) * 10

        bits_bufs = (bits_v0, bits_v1)
        out_bufs = (out_v0, out_v1)
        in_sems = (in_sem0, in_sem1)
        out_sems = (out_sem0, out_sem1)

        def compute_row(bits_v, out_v):
            @plsc.parallel_loop(0, groups, unroll=4)
            def _grp(g):
                idx0 = ten_iota + g * (10 * _LANES)
                full = plsc.load_gather(bits_v, [idx0])
                for j in range(1, 10):
                    bj = plsc.load_gather(bits_v, [idx0 + j])
                    full = full * 2 + bj
                re = plsc.load_gather(const_v, [full * 2])
                im = plsc.load_gather(const_v, [full * 2 + 1])
                out_v[pl.ds(g * _LANES, _LANES)] = re
                out_v[pl.ds(syms + g * _LANES, _LANES)] = im

        pltpu.async_copy(bits_hbm.at[row0], bits_v0, in_sem0)
        pltpu.async_copy(bits_hbm.at[row0 + 1], bits_v1, in_sem1)

        def pair_body(p, carry):
            for b in range(2):
                r = 2 * p + b
                row = row0 + r
                pltpu.make_async_copy(
                    bits_hbm.at[row], bits_bufs[b], in_sems[b]).wait()

                @pl.when(p > 0)
                def _():
                    pltpu.make_async_copy(
                        out_bufs[b], out_hbm.at[row - 2], out_sems[b]).wait()

                compute_row(bits_bufs[b], out_bufs[b])

                @pl.when(r + 2 < rows_per_w)
                def _():
                    pltpu.async_copy(
                        bits_hbm.at[row + 2], bits_bufs[b], in_sems[b])

                pltpu.async_copy(out_bufs[b], out_hbm.at[row], out_sems[b])
            return carry

        lax.fori_loop(0, rows_per_w // 2, pair_body, 0)

        last = row0 + rows_per_w
        pltpu.make_async_copy(out_v0, out_hbm.at[last - 2], out_sem0).wait()
        pltpu.make_async_copy(out_v1, out_hbm.at[last - 1], out_sem1).wait()

    return qam_kernel


def kernel(bits, constellation):
    bits = bits.astype(jnp.int32)
    batch, nbits = bits.shape
    syms = nbits // 10
    const_flat = constellation.astype(jnp.float32).reshape(-1)
    out = _build(batch, nbits)(bits, const_flat)
    return out.reshape(batch, 2, syms)


# speedup vs baseline: 171.9928x; 1.4938x over previous
import functools

import jax
import jax.numpy as jnp
from jax import lax
from jax.experimental import pallas as pl
from jax.experimental.pallas import tpu as pltpu
from jax.experimental.pallas import tpu_sc as plsc

_LANES = 16


@functools.lru_cache(maxsize=None)
def _build(batch, nbits):
    info = plsc.get_sparse_core_info()
    num_cores, num_subcores = info.num_cores, info.num_subcores
    nw = num_cores * num_subcores
    syms = nbits // 10
    assert batch % (2 * nw) == 0
    rows_per_w = batch // nw
    groups = syms // _LANES

    mesh = plsc.VectorSubcoreMesh(core_axis_name="c", subcore_axis_name="s")

    @functools.partial(
        pl.kernel,
        mesh=mesh,
        out_type=jax.ShapeDtypeStruct((batch, 2, syms), jnp.float32),
        compiler_params=pltpu.CompilerParams(needs_layout_passes=False),
        scratch_types=[
            pltpu.VMEM((nbits,), jnp.int32),
            pltpu.VMEM((nbits,), jnp.int32),
            pltpu.VMEM((2, syms), jnp.float32),
            pltpu.VMEM((2, syms), jnp.float32),
            pltpu.VMEM((2048,), jnp.float32),
            pltpu.SemaphoreType.DMA,
            pltpu.SemaphoreType.DMA,
            pltpu.SemaphoreType.DMA,
            pltpu.SemaphoreType.DMA,
        ],
    )
    def qam_kernel(bits_hbm, const_hbm, out_hbm, bits_v0, bits_v1, out_v0,
                   out_v1, const_v, in_sem0, in_sem1, out_sem0, out_sem1):
        wid = lax.axis_index("s") * num_cores + lax.axis_index("c")
        row0 = wid * rows_per_w
        pltpu.sync_copy(const_hbm, const_v)
        ten_iota = lax.iota(jnp.int32, _LANES) * 10

        bits_bufs = (bits_v0, bits_v1)
        out_bufs = (out_v0, out_v1)
        in_sems = (in_sem0, in_sem1)
        out_sems = (out_sem0, out_sem1)

        def compute_row(bits_v, out_v):
            @plsc.parallel_loop(0, groups, unroll=4)
            def _grp(g):
                idx0 = ten_iota + g * (10 * _LANES)
                full = plsc.load_gather(bits_v, [idx0])
                for j in range(1, 10):
                    bj = plsc.load_gather(bits_v, [idx0 + j])
                    full = full * 2 + bj
                re = plsc.load_gather(const_v, [full * 2])
                im = plsc.load_gather(const_v, [full * 2 + 1])
                out_v[0, pl.ds(g * _LANES, _LANES)] = re
                out_v[1, pl.ds(g * _LANES, _LANES)] = im

        pltpu.async_copy(bits_hbm.at[row0], bits_v0, in_sem0)
        pltpu.async_copy(bits_hbm.at[row0 + 1], bits_v1, in_sem1)

        def pair_body(p, carry):
            for b in range(2):
                r = 2 * p + b
                row = row0 + r
                pltpu.make_async_copy(
                    bits_hbm.at[row], bits_bufs[b], in_sems[b]).wait()

                @pl.when(p > 0)
                def _():
                    pltpu.make_async_copy(
                        out_bufs[b], out_hbm.at[row - 2], out_sems[b]).wait()

                compute_row(bits_bufs[b], out_bufs[b])

                @pl.when(r + 2 < rows_per_w)
                def _():
                    pltpu.async_copy(
                        bits_hbm.at[row + 2], bits_bufs[b], in_sems[b])

                pltpu.async_copy(out_bufs[b], out_hbm.at[row], out_sems[b])
            return carry

        lax.fori_loop(0, rows_per_w // 2, pair_body, 0)

        last = row0 + rows_per_w
        pltpu.make_async_copy(out_v0, out_hbm.at[last - 2], out_sem0).wait()
        pltpu.make_async_copy(out_v1, out_hbm.at[last - 1], out_sem1).wait()

    return qam_kernel


def kernel(bits, constellation):
    bits = bits.astype(jnp.int32)
    batch, nbits = bits.shape
    const_flat = constellation.astype(jnp.float32).reshape(-1)
    return _build(batch, nbits)(bits, const_flat)
